# baseline (speedup 1.0000x reference)
import jax
import jax.numpy as jnp
from jax.experimental import pallas as pl
from jax.experimental.pallas import tpu as pltpu

B = 4
S = 8192
D = 4096


def _body(hs_hbm, mask_ref, out_ref, sem):
  copies = []
  for b in range(B):
    cnt = jnp.sum(mask_ref[pl.ds(b, 1), :])
    idx = jnp.maximum(cnt - 1, 0)
    c = pltpu.make_async_copy(
        hs_hbm.at[b, pl.ds(idx, 1), :], out_ref.at[pl.ds(b, 1), :], sem)
    c.start()
    copies.append(c)
  for c in copies:
    c.wait()


@jax.jit
def _pooled(hidden_states, mask):
  f = pl.pallas_call(
      _body,
      out_shape=jax.ShapeDtypeStruct((B, D), jnp.float32),
      in_specs=[
          pl.BlockSpec(memory_space=pl.ANY),
          pl.BlockSpec((B, S), lambda: (0, 0)),
      ],
      out_specs=pl.BlockSpec(memory_space=pl.ANY),
      scratch_shapes=[pltpu.SemaphoreType.DMA],
  )
  return f(hidden_states, mask)


def kernel(hidden_states, mask):
  return _pooled(hidden_states, mask.astype(jnp.int32))


# speedup vs baseline: 1.5809x; 1.5809x over previous
import jax
import jax.numpy as jnp
from jax.experimental import pallas as pl
from jax.experimental.pallas import tpu as pltpu

B = 4
S = 8192
D = 4096


def _body(hs_hbm, mask_hbm, out_ref, mask_v, sem, rsem):
  mc = pltpu.make_async_copy(mask_hbm, mask_v, sem)
  mc.start()
  mc.wait()
  copies = []
  for b in range(B):
    cnt = jnp.sum(mask_v[pl.ds(b, 1), :])
    idx = jnp.maximum(cnt - 1, 0)
    c = pltpu.make_async_copy(
        hs_hbm.at[b, pl.ds(idx, 1), :], out_ref.at[pl.ds(b, 1), :], rsem)
    c.start()
    copies.append(c)
  for c in copies:
    c.wait()


@jax.jit
def _pooled(hidden_states, mask):
  f = pl.pallas_call(
      _body,
      out_shape=jax.ShapeDtypeStruct((B, D), jnp.float32),
      in_specs=[
          pl.BlockSpec(memory_space=pl.ANY),
          pl.BlockSpec(memory_space=pl.ANY),
      ],
      out_specs=pl.BlockSpec((B, D), lambda: (0, 0)),
      scratch_shapes=[
          pltpu.VMEM((B, S), jnp.int32),
          pltpu.SemaphoreType.DMA,
          pltpu.SemaphoreType.DMA,
      ],
  )
  return f(hidden_states, mask)


def kernel(hidden_states, mask):
  return _pooled(hidden_states, mask.astype(jnp.int32))


# speedup vs baseline: 2.2441x; 1.4195x over previous
import jax
import jax.numpy as jnp
from jax.experimental import pallas as pl
from jax.experimental.pallas import tpu as pltpu

B = 4
S = 8192
D = 4096


def _body(hs_hbm, mask_hbm, out_ref, mask_v, msem, ssem, csem):
  mc = pltpu.make_async_copy(mask_hbm, mask_v, msem)
  mc.start()
  spec = []
  for b in range(B):
    c = pltpu.make_async_copy(
        hs_hbm.at[b, pl.ds(S - 1, 1), :], out_ref.at[pl.ds(b, 1), :], ssem)
    c.start()
    spec.append(c)
  mc.wait()

  idxs = []
  for b in range(B):
    cnt = jnp.sum(mask_v[pl.ds(b, 1), :])
    idxs.append(jnp.maximum(cnt - 1, 0))
  for c in spec:
    c.wait()

  for b in range(B):
    idx = idxs[b]

    @pl.when(idx != S - 1)
    def _fix(b=b, idx=idx):
      c = pltpu.make_async_copy(
          hs_hbm.at[b, pl.ds(idx, 1), :], out_ref.at[pl.ds(b, 1), :], csem)
      c.start()
      c.wait()


@jax.jit
def _pooled(hidden_states, mask):
  f = pl.pallas_call(
      _body,
      out_shape=jax.ShapeDtypeStruct((B, D), jnp.float32),
      in_specs=[
          pl.BlockSpec(memory_space=pl.ANY),
          pl.BlockSpec(memory_space=pl.ANY),
      ],
      out_specs=pl.BlockSpec((B, D), lambda: (0, 0)),
      scratch_shapes=[
          pltpu.VMEM((B, S), jnp.int32),
          pltpu.SemaphoreType.DMA,
          pltpu.SemaphoreType.DMA,
          pltpu.SemaphoreType.DMA,
      ],
  )
  return f(hidden_states, mask)


def kernel(hidden_states, mask):
  return _pooled(hidden_states, mask.astype(jnp.int32))
